# Initial kernel scaffold; baseline (speedup 1.0000x reference)
import jax
import jax.numpy as jnp
from jax.experimental import pallas as pl


def kernel(x, adj_matrix, W1, b1, g1, be1, W2, b2, g2, be2):
    raise NotImplementedError("write your pallas kernel here")


# speedup vs baseline: 2.1032x; 2.1032x over previous
import jax
import jax.numpy as jnp
from jax.experimental import pallas as pl

N = 1024
EPS = 1e-5


def _gcn_body(x_ref, adj_ref, W1_ref, b1_ref, g1_ref, be1_ref,
              W2_ref, b2_ref, g2_ref, be2_ref, out_ref):
    adj = adj_ref[...]
    rows = jax.lax.broadcasted_iota(jnp.int32, (N, N), 0)
    cols = jax.lax.broadcasted_iota(jnp.int32, (N, N), 1)
    a_hat = jnp.where(rows == cols, 1.0, adj)
    deg = jnp.sum(a_hat, axis=1, keepdims=True)
    dinv = jax.lax.rsqrt(deg)

    def layer(h, W_ref, b_ref, g_ref, be_ref, relu):
        z = jnp.dot(h, W_ref[...], preferred_element_type=jnp.float32)
        z = z * dinv
        h2 = jnp.dot(a_hat, z, preferred_element_type=jnp.float32)
        h2 = h2 * dinv + b_ref[...]
        mu = jnp.mean(h2, axis=0, keepdims=True)
        var = jnp.mean((h2 - mu) ** 2, axis=0, keepdims=True)
        h2 = g_ref[...] * (h2 - mu) * jax.lax.rsqrt(var + EPS) + be_ref[...]
        if relu:
            h2 = jnp.maximum(h2, 0.0)
        return h2

    h = layer(x_ref[...], W1_ref, b1_ref, g1_ref, be1_ref, relu=True)
    out_ref[...] = layer(h, W2_ref, b2_ref, g2_ref, be2_ref, relu=False)


def kernel(x, adj_matrix, W1, b1, g1, be1, W2, b2, g2, be2):
    vecs = [v.reshape(1, -1) for v in (b1, g1, be1, b2, g2, be2)]
    return pl.pallas_call(
        _gcn_body,
        out_shape=jax.ShapeDtypeStruct((N, W2.shape[1]), jnp.float32),
    )(x, adj_matrix, W1, vecs[0], vecs[1], vecs[2], W2, vecs[3], vecs[4], vecs[5])
